# Initial kernel scaffold; baseline (speedup 1.0000x reference)
import jax
import jax.numpy as jnp
from jax.experimental import pallas as pl


def kernel(subj, rel, obj, ent_re, ent_im, rel_re, rel_im):
    raise NotImplementedError("write your pallas kernel here")


# speedup vs baseline: 1.2979x; 1.2979x over previous
import jax
import jax.numpy as jnp
from jax import lax
from jax.experimental import pallas as pl
from jax.experimental.pallas import tpu as pltpu
from jax.experimental.pallas import tpu_sc as plsc

_D = 64
_B = 16384
_NC = 2
_NS = 16
_NW = _NC * _NS
_BPW = _B // _NW
_CHUNK = 128
_NCHUNK = _BPW // _CHUNK
_NBUF = 2


def _tec_kernel(subj_h, rel_h, obj_h, ent_re_h, ent_im_h, rel_re_h, rel_im_h,
                out_h,
                subj_v, rel_v, obj_v,
                bufs_a, bufs_b, stage_v, out_v, sem_a, sem_b):
    wid = lax.axis_index("s") * _NC + lax.axis_index("c")
    base = wid * _BPW

    for c in range(_NCHUNK):
        pltpu.sync_copy(subj_h.at[pl.ds(base + c * _CHUNK, _CHUNK)], subj_v.at[c])
        pltpu.sync_copy(rel_h.at[pl.ds(base + c * _CHUNK, _CHUNK)], rel_v.at[c])
        pltpu.sync_copy(obj_h.at[pl.ds(base + c * _CHUNK, _CHUNK)], obj_v.at[c])

    bufs = (bufs_a, bufs_b)
    sems = (sem_a, sem_b)

    def fire(c):
        p = c % _NBUF
        sre, sim, ore, oim, rre, rim = bufs[p]
        sem = sems[p]
        return [
            pltpu.async_copy(ent_re_h.at[subj_v.at[c]], sre, sem),
            pltpu.async_copy(ent_im_h.at[subj_v.at[c]], sim, sem),
            pltpu.async_copy(ent_re_h.at[obj_v.at[c]], ore, sem),
            pltpu.async_copy(ent_im_h.at[obj_v.at[c]], oim, sem),
            pltpu.async_copy(rel_re_h.at[rel_v.at[c]], rre, sem),
            pltpu.async_copy(rel_im_h.at[rel_v.at[c]], rim, sem),
        ]

    iota = lax.iota(jnp.int32, 16)
    scatter_idx = iota * _CHUNK
    descs = fire(0)

    for c in range(_NCHUNK):
        next_descs = fire(c + 1) if c + 1 < _NCHUNK else None
        for d_ in descs:
            d_.wait()
        sre, sim, ore, oim, rre, rim = bufs[c % _NBUF]

        def triple_body(i, _, sre=sre, sim=sim, ore=ore, oim=oim,
                        rre=rre, rim=rim):
            acc = jnp.zeros((16,), jnp.float32)
            for k in range(_D // 16):
                sl = pl.ds(k * 16, 16)
                a = sre[i, sl]
                b = sim[i, sl]
                x = ore[i, sl]
                y = oim[i, sl]
                p = rre[i, sl]
                q = rim[i, sl]
                u = p * x + q * y
                v = p * y - q * x
                acc = acc + a * u + b * v
            plsc.store_scatter(stage_v, [scatter_idx + i], acc)
            return 0

        lax.fori_loop(0, _CHUNK, triple_body, 0)

        def group_body(g, _, c=c):
            acc = stage_v[pl.ds(g * 16, 16)]
            for k in range(1, 16):
                acc = acc + stage_v[pl.ds(k * _CHUNK + g * 16, 16)]
            out_v[pl.ds(c * _CHUNK + g * 16, 16)] = acc
            return 0

        lax.fori_loop(0, _CHUNK // 16, group_body, 0)
        descs = next_descs

    pltpu.sync_copy(out_v, out_h.at[pl.ds(base, _BPW)])


@jax.jit
def kernel(subj, rel, obj, ent_re, ent_im, rel_re, rel_im):
    mesh = plsc.VectorSubcoreMesh(core_axis_name="c", subcore_axis_name="s")
    row_buf = lambda: pltpu.VMEM((_CHUNK, _D), jnp.float32)
    run = pl.kernel(
        _tec_kernel,
        out_type=jax.ShapeDtypeStruct((_B,), jnp.float32),
        mesh=mesh,
        compiler_params=pltpu.CompilerParams(
            needs_layout_passes=False, use_tc_tiling_on_sc=False),
        scratch_types=[
            pltpu.VMEM((_NCHUNK, _CHUNK), jnp.int32),
            pltpu.VMEM((_NCHUNK, _CHUNK), jnp.int32),
            pltpu.VMEM((_NCHUNK, _CHUNK), jnp.int32),
            [row_buf() for _ in range(6)],
            [row_buf() for _ in range(6)],
            pltpu.VMEM((16 * _CHUNK,), jnp.float32),
            pltpu.VMEM((_BPW,), jnp.float32),
            pltpu.SemaphoreType.DMA,
            pltpu.SemaphoreType.DMA,
        ],
    )
    return run(subj, rel, obj, ent_re, ent_im, rel_re, rel_im)


# speedup vs baseline: 1.4754x; 1.1368x over previous
import jax
import jax.numpy as jnp
from jax import lax
from jax.experimental import pallas as pl
from jax.experimental.pallas import tpu as pltpu
from jax.experimental.pallas import tpu_sc as plsc

_D = 64
_W = 2 * _D
_B = 16384
_NC = 2
_NS = 16
_NW = _NC * _NS
_BPW = _B // _NW
_CHUNK = 128
_NCHUNK = _BPW // _CHUNK
_NBUF = 2


def _tec_kernel(subj_h, rel_h, obj_h, entcat_h, relcat_h,
                out_h,
                subj_v, rel_v, obj_v,
                bufs_a, bufs_b, stage_v, out_v, sem_a, sem_b):
    wid = lax.axis_index("s") * _NC + lax.axis_index("c")
    base = wid * _BPW

    for c in range(_NCHUNK):
        pltpu.sync_copy(subj_h.at[pl.ds(base + c * _CHUNK, _CHUNK)], subj_v.at[c])
        pltpu.sync_copy(rel_h.at[pl.ds(base + c * _CHUNK, _CHUNK)], rel_v.at[c])
        pltpu.sync_copy(obj_h.at[pl.ds(base + c * _CHUNK, _CHUNK)], obj_v.at[c])

    bufs = (bufs_a, bufs_b)
    sems = (sem_a, sem_b)

    def fire(c):
        p = c % _NBUF
        s_b, o_b, r_b = bufs[p]
        sem = sems[p]
        return [
            pltpu.async_copy(entcat_h.at[subj_v.at[c]], s_b, sem),
            pltpu.async_copy(entcat_h.at[obj_v.at[c]], o_b, sem),
            pltpu.async_copy(relcat_h.at[rel_v.at[c]], r_b, sem),
        ]

    iota = lax.iota(jnp.int32, 16)
    scatter_idx = iota * _CHUNK
    descs = fire(0)

    for c in range(_NCHUNK):
        next_descs = fire(c + 1) if c + 1 < _NCHUNK else None
        for d_ in descs:
            d_.wait()
        s_b, o_b, r_b = bufs[c % _NBUF]

        def triple_body(i, _, s_b=s_b, o_b=o_b, r_b=r_b):
            acc = jnp.zeros((16,), jnp.float32)
            for k in range(_D // 16):
                re_sl = pl.ds(k * 16, 16)
                im_sl = pl.ds(_D + k * 16, 16)
                a = s_b[i, re_sl]
                b = s_b[i, im_sl]
                x = o_b[i, re_sl]
                y = o_b[i, im_sl]
                p = r_b[i, re_sl]
                q = r_b[i, im_sl]
                u = p * x + q * y
                v = p * y - q * x
                acc = acc + a * u + b * v
            plsc.store_scatter(stage_v, [scatter_idx + i], acc)
            return 0

        lax.fori_loop(0, _CHUNK, triple_body, 0)

        def group_body(g, _, c=c):
            acc = stage_v[pl.ds(g * 16, 16)]
            for k in range(1, 16):
                acc = acc + stage_v[pl.ds(k * _CHUNK + g * 16, 16)]
            out_v[pl.ds(c * _CHUNK + g * 16, 16)] = acc
            return 0

        lax.fori_loop(0, _CHUNK // 16, group_body, 0)
        descs = next_descs

    pltpu.sync_copy(out_v, out_h.at[pl.ds(base, _BPW)])


@jax.jit
def kernel(subj, rel, obj, ent_re, ent_im, rel_re, rel_im):
    entcat = jnp.concatenate([ent_re, ent_im], axis=1)
    relcat = jnp.concatenate([rel_re, rel_im], axis=1)
    mesh = plsc.VectorSubcoreMesh(core_axis_name="c", subcore_axis_name="s")
    row_buf = lambda: pltpu.VMEM((_CHUNK, _W), jnp.float32)
    run = pl.kernel(
        _tec_kernel,
        out_type=jax.ShapeDtypeStruct((_B,), jnp.float32),
        mesh=mesh,
        compiler_params=pltpu.CompilerParams(
            needs_layout_passes=False, use_tc_tiling_on_sc=True),
        scratch_types=[
            pltpu.VMEM((_NCHUNK, _CHUNK), jnp.int32),
            pltpu.VMEM((_NCHUNK, _CHUNK), jnp.int32),
            pltpu.VMEM((_NCHUNK, _CHUNK), jnp.int32),
            [row_buf() for _ in range(3)],
            [row_buf() for _ in range(3)],
            pltpu.VMEM((16 * _CHUNK,), jnp.float32),
            pltpu.VMEM((_BPW,), jnp.float32),
            pltpu.SemaphoreType.DMA,
            pltpu.SemaphoreType.DMA,
        ],
    )
    return run(subj, rel, obj, entcat, relcat)
